# Initial kernel scaffold; baseline (speedup 1.0000x reference)
import jax
import jax.numpy as jnp
from jax.experimental import pallas as pl


def kernel(cat, cont, emb_tables, cont_W, cont_b):
    raise NotImplementedError("write your pallas kernel here")


# speedup vs baseline: 2.5142x; 2.5142x over previous
import functools

import jax
import jax.numpy as jnp
from jax import lax
from jax.experimental import pallas as pl
from jax.experimental.pallas import tpu as pltpu
from jax.experimental.pallas import tpu_sc as plsc

B, L, C, F, V, D = 1024, 50, 26, 13, 100000, 32
N = B * L
OUT_C = C + F

_info = plsc.get_sparse_core_info()
NC, NS = _info.num_cores, _info.num_subcores
NW = NC * NS
TOK_PER_W = N // NW
T = 64
CHUNKS = TOK_PER_W // T


def _make_sc_kernel():
    mesh = plsc.VectorSubcoreMesh(core_axis_name="c", subcore_axis_name="s")

    @functools.partial(
        pl.kernel,
        mesh=mesh,
        compiler_params=pltpu.CompilerParams(use_tc_tiling_on_sc=False),
        out_type=jax.ShapeDtypeStruct((N, OUT_C, D), jnp.float32),
        scratch_types=[
            pltpu.VMEM((T, C), jnp.int32),
            pltpu.VMEM((T, OUT_C, D), jnp.float32),
            pltpu.VMEM((T, 16), jnp.float32),
            pltpu.VMEM((F, D), jnp.float32),
            pltpu.VMEM((F, D), jnp.float32),
            pltpu.SemaphoreType.DMA,
        ],
    )
    def sc_kernel(cat_hbm, cont_hbm, tab_hbm, w_hbm, b_hbm, out_hbm,
                  idx_v, comb_v, cin_v, w_v, b_v, gsem):
        wid = lax.axis_index("s") * NC + lax.axis_index("c")
        base = wid * TOK_PER_W

        pltpu.sync_copy(w_hbm, w_v)
        pltpu.sync_copy(b_hbm, b_v)

        def chunk_body(g, carry):
            n0 = base + g * T
            pltpu.sync_copy(cat_hbm.at[pl.ds(n0, T)], idx_v)
            pltpu.sync_copy(cont_hbm.at[pl.ds(n0, T)], cin_v)

            def fire(t, c):
                pltpu.async_copy(
                    tab_hbm.at[idx_v.at[t]], comb_v.at[t, pl.ds(0, C), :], gsem
                )
                return c
            lax.fori_loop(0, T, fire, 0)

            def cont_body(t, c):
                xrow = cin_v[t]
                for f in range(F):
                    xv = jnp.full((16,), xrow[f], dtype=jnp.float32)
                    for h in range(D // 16):
                        wv = w_v[f, pl.ds(h * 16, 16)]
                        bv = b_v[f, pl.ds(h * 16, 16)]
                        comb_v[t, C + f, pl.ds(h * 16, 16)] = xv * wv + bv
                return c
            lax.fori_loop(0, T, cont_body, 0)

            def drain(t, c):
                pltpu.make_async_copy(
                    tab_hbm.at[idx_v.at[t]], comb_v.at[t, pl.ds(0, C), :], gsem
                ).wait()
                return c
            lax.fori_loop(0, T, drain, 0)

            pltpu.sync_copy(comb_v, out_hbm.at[pl.ds(n0, T)])
            return carry

        lax.fori_loop(0, CHUNKS, chunk_body, 0)

    return sc_kernel


_sc_kernel = _make_sc_kernel()


def kernel(cat, cont, emb_tables, cont_W, cont_b):
    offs = (jnp.arange(C, dtype=jnp.int32) * V)[None, :]
    cat2 = cat.reshape(N, C).astype(jnp.int32) + offs
    cont2 = jnp.pad(cont.reshape(N, F), ((0, 0), (0, 16 - F)))
    tab = emb_tables.reshape(C * V, D)
    return _sc_kernel(cat2, cont2, tab, cont_W, cont_b)


# speedup vs baseline: 5.5831x; 2.2206x over previous
import functools

import jax
import jax.numpy as jnp
from jax import lax
from jax.experimental import pallas as pl
from jax.experimental.pallas import tpu as pltpu
from jax.experimental.pallas import tpu_sc as plsc

B, L, C, F, V, D = 1024, 50, 26, 13, 100000, 32
N = B * L
OUT_C = C + F
VP = 100096
VB = VP // 128
NB = N // 128
KCH = 8
RCH = NB // KCH

_info = plsc.get_sparse_core_info()
NC, NS = _info.num_cores, _info.num_subcores
NW = NC * NS


def _make_sc_kernel():
    mesh = plsc.VectorSubcoreMesh(core_axis_name="c", subcore_axis_name="s")

    @functools.partial(
        pl.kernel,
        mesh=mesh,
        compiler_params=pltpu.CompilerParams(
            use_tc_tiling_on_sc=False, needs_layout_passes=False),
        out_type=jax.ShapeDtypeStruct((OUT_C, 4, NB, 8, 128), jnp.float32),
        scratch_types=[
            pltpu.VMEM((VB, 128), jnp.float32),
            pltpu.VMEM((RCH, 128), jnp.int32),
            pltpu.VMEM((RCH, 128), jnp.float32),
            pltpu.VMEM((RCH, 128), jnp.float32),
            pltpu.VMEM((16,), jnp.float32),
            pltpu.VMEM((16,), jnp.float32),
        ],
    )
    def sc_kernel(cat_hbm, cont_hbm, tab_hbm, w_hbm, b_hbm, out_hbm,
                  src_v, idx_v, x_v, o_v, wr_v, br_v):
        d = lax.axis_index("s") * NC + lax.axis_index("c")
        db = d // 8
        dm = d % 8

        pltpu.sync_copy(w_hbm.at[d], wr_v)
        pltpu.sync_copy(b_hbm.at[d], br_v)

        def cat_task(c, carry):
            pltpu.sync_copy(tab_hbm.at[c, db, :, dm, :], src_v)

            def chunk(k, cc):
                pltpu.sync_copy(cat_hbm.at[c, pl.ds(k * RCH, RCH), :], idx_v)

                def row(r, rc):
                    for h in range(8):
                        iv = idx_v[r, pl.ds(h * 16, 16)]
                        vb = jax.lax.shift_right_logical(iv, 7)
                        vm = jax.lax.bitwise_and(iv, 127)
                        o_v[r, pl.ds(h * 16, 16)] = plsc.load_gather(
                            src_v, [vb, vm])
                    return rc
                lax.fori_loop(0, RCH, row, 0)
                pltpu.sync_copy(
                    o_v, out_hbm.at[c, db, pl.ds(k * RCH, RCH), dm, :])
                return cc
            lax.fori_loop(0, KCH, chunk, 0)
            return carry
        lax.fori_loop(0, C, cat_task, 0)

        wrow = wr_v[...]
        brow = br_v[...]
        for f in range(F):
            ws = jnp.full((16,), wrow[f], dtype=jnp.float32)
            bs = jnp.full((16,), brow[f], dtype=jnp.float32)

            def chunk_f(k, cc, f=f, ws=ws, bs=bs):
                pltpu.sync_copy(cont_hbm.at[f, pl.ds(k * RCH, RCH), :], x_v)

                def row(r, rc):
                    for h in range(8):
                        xv = x_v[r, pl.ds(h * 16, 16)]
                        o_v[r, pl.ds(h * 16, 16)] = xv * ws + bs
                    return rc
                lax.fori_loop(0, RCH, row, 0)
                pltpu.sync_copy(
                    o_v, out_hbm.at[C + f, db, pl.ds(k * RCH, RCH), dm, :])
                return cc
            lax.fori_loop(0, KCH, chunk_f, 0)

    return sc_kernel


_sc_kernel = _make_sc_kernel()


def kernel(cat, cont, emb_tables, cont_W, cont_b):
    catT = cat.reshape(N, C).T.reshape(C, NB, 128)
    contT = cont.reshape(N, F).T.reshape(F, NB, 128)
    tabT = emb_tables.transpose(0, 2, 1)
    tabP = jnp.pad(tabT, ((0, 0), (0, 0), (0, VP - V)))
    tab5 = tabP.reshape(C, 4, 8, VB, 128).transpose(0, 1, 3, 2, 4)
    wT = jnp.zeros((32, 16), jnp.float32).at[:, :F].set(cont_W.T)
    bT = jnp.zeros((32, 16), jnp.float32).at[:, :F].set(cont_b.T)
    out5 = _sc_kernel(catT, contT, tab5, wT, bT)
    return out5.transpose(2, 4, 0, 1, 3).reshape(N, OUT_C, D)
